# Initial kernel scaffold; baseline (speedup 1.0000x reference)
import jax
import jax.numpy as jnp
from jax.experimental import pallas as pl


def kernel(pred, target, input_img):
    raise NotImplementedError("write your pallas kernel here")


# speedup vs baseline: 2.5507x; 2.5507x over previous
import functools

import jax
import jax.numpy as jnp
from jax import lax
from jax.experimental import pallas as pl
from jax.experimental.pallas import tpu as pltpu
from jax.experimental.pallas import tpu_sc as plsc

_NUM_BINS = 32
_NCH = 3
_L = 16
_LANE_STRIDE = 128
_ACC_SIZE = _L * _LANE_STRIDE
_NSEG = _NCH * _NUM_BINS

_TOTAL = 8 * 3 * 512 * 512
_NUNITS = 96
_UNIT = _TOTAL // _NUNITS
_CHUNK = 16384
_CHUNKS_PER_UNIT = _UNIT // _CHUNK
_NW = 32
_UNITS_PER_W = _NUNITS // _NW


def _sc_body(x_hbm, p_hbm, t_hbm, sums_out, cnts_out,
             xb, pb, tb, acc_s, acc_c, row_s, row_c, sem):
    wid = lax.axis_index("s") * 2 + lax.axis_index("c")

    zeros = jnp.zeros((_L,), jnp.float32)
    ones = jnp.ones((_L,), jnp.float32)
    lane_base = lax.iota(jnp.int32, _L) * _LANE_STRIDE

    def _zero(k, _):
        acc_s[pl.ds(k * _L, _L)] = zeros
        acc_c[pl.ds(k * _L, _L)] = zeros
        return 0
    lax.fori_loop(0, _ACC_SIZE // _L, _zero, 0)

    for j in range(_UNITS_PER_W):
        u = wid * _UNITS_PER_W + j
        ch = lax.rem(lax.div(u, 4), _NCH)
        laneoff = lane_base + ch * _NUM_BINS
        for cidx in range(_CHUNKS_PER_UNIT):
            base = u * _UNIT + cidx * _CHUNK
            pltpu.sync_copy(x_hbm.at[pl.ds(base, _CHUNK)], xb)
            pltpu.sync_copy(p_hbm.at[pl.ds(base, _CHUNK)], pb)
            pltpu.sync_copy(t_hbm.at[pl.ds(base, _CHUNK)], tb)

            def _vec(v, _):
                sl = pl.ds(v * _L, _L)
                x = xb[sl]
                d = pb[sl] - tb[sl]
                bi = (x * jnp.float32(_NUM_BINS)).astype(jnp.int32)
                bi = jnp.minimum(jnp.maximum(bi, 0), _NUM_BINS - 1)
                idx = laneoff + bi
                plsc.addupdate_scatter(acc_s, [idx], d)
                plsc.addupdate_scatter(acc_c, [idx], ones)
                return 0
            lax.fori_loop(0, _CHUNK // _L, _vec, 0, unroll=4)

    for k in range(_NSEG // _L):
        ts = zeros
        tc = zeros
        for lane in range(_L):
            off = lane * _LANE_STRIDE + k * _L
            ts = ts + acc_s[pl.ds(off, _L)]
            tc = tc + acc_c[pl.ds(off, _L)]
        row_s[pl.ds(k * _L, _L)] = ts
        row_c[pl.ds(k * _L, _L)] = tc

    pltpu.sync_copy(row_s, sums_out.at[wid])
    pltpu.sync_copy(row_c, cnts_out.at[wid])


_sc_call = functools.partial(
    pl.kernel,
    out_type=(jax.ShapeDtypeStruct((_NW, _NSEG), jnp.float32),
              jax.ShapeDtypeStruct((_NW, _NSEG), jnp.float32)),
    mesh=plsc.VectorSubcoreMesh(core_axis_name="c", subcore_axis_name="s"),
    compiler_params=pltpu.CompilerParams(needs_layout_passes=False),
    scratch_types=(
        pltpu.VMEM((_CHUNK,), jnp.float32),
        pltpu.VMEM((_CHUNK,), jnp.float32),
        pltpu.VMEM((_CHUNK,), jnp.float32),
        pltpu.VMEM((_ACC_SIZE,), jnp.float32),
        pltpu.VMEM((_ACC_SIZE,), jnp.float32),
        pltpu.VMEM((_NSEG,), jnp.float32),
        pltpu.VMEM((_NSEG,), jnp.float32),
        pltpu.SemaphoreType.DMA,
    ),
)(_sc_body)


def _tc_body(s_ref, c_ref, o_ref):
    s = jnp.sum(s_ref[...], axis=0, keepdims=True)
    c = jnp.sum(c_ref[...], axis=0, keepdims=True)
    nonempty = c > 0.0
    val = jnp.where(nonempty, jnp.abs(s) / jnp.where(nonempty, c, 1.0), 0.0)
    o_ref[...] = jnp.full((1, 1), jnp.sum(val) / jnp.float32(_NSEG))


def kernel(pred, target, input_img):
    x = input_img.reshape(-1)
    p = pred.reshape(-1)
    t = target.reshape(-1)
    sums, cnts = _sc_call(x, p, t)
    loss = pl.pallas_call(
        _tc_body,
        out_shape=jax.ShapeDtypeStruct((1, 1), jnp.float32),
    )(sums, cnts)
    return loss[0, 0]


# speedup vs baseline: 2.9350x; 1.1507x over previous
import functools

import jax
import jax.numpy as jnp
from jax import lax
from jax.experimental import pallas as pl
from jax.experimental.pallas import tpu as pltpu
from jax.experimental.pallas import tpu_sc as plsc

_NUM_BINS = 32
_NCH = 3
_L = 16
_LANE_STRIDE = 128
_ACC_SIZE = _L * _LANE_STRIDE
_NSEG = _NCH * _NUM_BINS

_TOTAL = 8 * 3 * 512 * 512
_NUNITS = 96
_UNIT = _TOTAL // _NUNITS
_CHUNK = 16384
_CHUNKS_PER_UNIT = _UNIT // _CHUNK
_NW = 32
_UNITS_PER_W = _NUNITS // _NW


def _sc_body(x_hbm, p_hbm, t_hbm, sums_out, cnts_out,
             xb0, xb1, pb0, pb1, tb0, tb1,
             acc_s, acc_c, row_s, row_c, sem0, sem1):
    wid = lax.axis_index("s") * 2 + lax.axis_index("c")

    zeros = jnp.zeros((_L,), jnp.float32)
    ones = jnp.ones((_L,), jnp.float32)
    lane_base = lax.iota(jnp.int32, _L) * _LANE_STRIDE

    def _zero(k, _):
        acc_s[pl.ds(k * _L, _L)] = zeros
        acc_c[pl.ds(k * _L, _L)] = zeros
        return 0
    lax.fori_loop(0, _ACC_SIZE // _L, _zero, 0)

    nsteps = _UNITS_PER_W * _CHUNKS_PER_UNIT
    bufs = ((xb0, pb0, tb0), (xb1, pb1, tb1))
    sems = (sem0, sem1)

    def _issue(step, slot):
        j, cidx = divmod(step, _CHUNKS_PER_UNIT)
        base = (wid * _UNITS_PER_W + j) * _UNIT + cidx * _CHUNK
        xs, ps, ts = bufs[slot]
        sem = sems[slot]
        return (
            pltpu.async_copy(x_hbm.at[pl.ds(base, _CHUNK)], xs, sem),
            pltpu.async_copy(p_hbm.at[pl.ds(base, _CHUNK)], ps, sem),
            pltpu.async_copy(t_hbm.at[pl.ds(base, _CHUNK)], ts, sem),
        )

    inflight = _issue(0, 0)
    for step in range(nsteps):
        slot = step % 2
        cur = inflight
        if step + 1 < nsteps:
            inflight = _issue(step + 1, 1 - slot)
        for c in cur:
            c.wait()
        j = step // _CHUNKS_PER_UNIT
        u = wid * _UNITS_PER_W + j
        ch = lax.rem(lax.div(u, 4), _NCH)
        laneoff = lane_base + ch * _NUM_BINS
        xs, ps, ts = bufs[slot]

        def _vec(v, _):
            sl = pl.ds(v * _L, _L)
            x = xs[sl]
            d = ps[sl] - ts[sl]
            bi = (x * jnp.float32(_NUM_BINS)).astype(jnp.int32)
            bi = jnp.minimum(jnp.maximum(bi, 0), _NUM_BINS - 1)
            idx = laneoff + bi
            plsc.addupdate_scatter(acc_s, [idx], d)
            plsc.addupdate_scatter(acc_c, [idx], ones)
            return 0
        lax.fori_loop(0, _CHUNK // _L, _vec, 0, unroll=8)

    for k in range(_NSEG // _L):
        ts = zeros
        tc = zeros
        for lane in range(_L):
            off = lane * _LANE_STRIDE + k * _L
            ts = ts + acc_s[pl.ds(off, _L)]
            tc = tc + acc_c[pl.ds(off, _L)]
        row_s[pl.ds(k * _L, _L)] = ts
        row_c[pl.ds(k * _L, _L)] = tc

    pltpu.sync_copy(row_s, sums_out.at[wid])
    pltpu.sync_copy(row_c, cnts_out.at[wid])


_sc_call = functools.partial(
    pl.kernel,
    out_type=(jax.ShapeDtypeStruct((_NW, _NSEG), jnp.float32),
              jax.ShapeDtypeStruct((_NW, _NSEG), jnp.float32)),
    mesh=plsc.VectorSubcoreMesh(core_axis_name="c", subcore_axis_name="s"),
    compiler_params=pltpu.CompilerParams(needs_layout_passes=False),
    scratch_types=(
        pltpu.VMEM((_CHUNK,), jnp.float32),
        pltpu.VMEM((_CHUNK,), jnp.float32),
        pltpu.VMEM((_CHUNK,), jnp.float32),
        pltpu.VMEM((_CHUNK,), jnp.float32),
        pltpu.VMEM((_CHUNK,), jnp.float32),
        pltpu.VMEM((_CHUNK,), jnp.float32),
        pltpu.VMEM((_ACC_SIZE,), jnp.float32),
        pltpu.VMEM((_ACC_SIZE,), jnp.float32),
        pltpu.VMEM((_NSEG,), jnp.float32),
        pltpu.VMEM((_NSEG,), jnp.float32),
        pltpu.SemaphoreType.DMA,
        pltpu.SemaphoreType.DMA,
    ),
)(_sc_body)


def _tc_body(s_ref, c_ref, o_ref):
    s = jnp.sum(s_ref[...], axis=0, keepdims=True)
    c = jnp.sum(c_ref[...], axis=0, keepdims=True)
    nonempty = c > 0.0
    val = jnp.where(nonempty, jnp.abs(s) / jnp.where(nonempty, c, 1.0), 0.0)
    o_ref[...] = jnp.full((1, 1), jnp.sum(val) / jnp.float32(_NSEG))


def kernel(pred, target, input_img):
    x = input_img.reshape(-1)
    p = pred.reshape(-1)
    t = target.reshape(-1)
    sums, cnts = _sc_call(x, p, t)
    loss = pl.pallas_call(
        _tc_body,
        out_shape=jax.ShapeDtypeStruct((1, 1), jnp.float32),
    )(sums, cnts)
    return loss[0, 0]


# speedup vs baseline: 3.1426x; 1.0707x over previous
import functools

import jax
import jax.numpy as jnp
from jax import lax
from jax.experimental import pallas as pl
from jax.experimental.pallas import tpu as pltpu
from jax.experimental.pallas import tpu_sc as plsc

_NUM_BINS = 32
_NCH = 3
_L = 16
_NSEG = _NCH * _NUM_BINS
_ACC_SIZE = _NSEG * _L

_TOTAL = 8 * 3 * 512 * 512
_NUNITS = 96
_UNIT = _TOTAL // _NUNITS
_CHUNK = 16384
_CHUNKS_PER_UNIT = _UNIT // _CHUNK
_NW = 32
_UNITS_PER_W = _NUNITS // _NW


def _sc_body(x_hbm, p_hbm, t_hbm, sums_out, cnts_out,
             xb0, xb1, pb0, pb1, tb0, tb1,
             acc_s, acc_c, row_s, row_c, sem0, sem1):
    wid = lax.axis_index("s") * 2 + lax.axis_index("c")

    zeros = jnp.zeros((_L,), jnp.float32)
    ones = jnp.ones((_L,), jnp.float32)
    lane_iota = lax.iota(jnp.int32, _L)

    def _zero(k, _):
        acc_s[pl.ds(k * _L, _L)] = zeros
        acc_c[pl.ds(k * _L, _L)] = zeros
        return 0
    lax.fori_loop(0, _ACC_SIZE // _L, _zero, 0)

    nsteps = _UNITS_PER_W * _CHUNKS_PER_UNIT
    bufs = ((xb0, pb0, tb0), (xb1, pb1, tb1))
    sems = (sem0, sem1)

    def _issue(step, slot):
        j, cidx = divmod(step, _CHUNKS_PER_UNIT)
        base = (wid * _UNITS_PER_W + j) * _UNIT + cidx * _CHUNK
        xs, ps, ts = bufs[slot]
        sem = sems[slot]
        return (
            pltpu.async_copy(x_hbm.at[pl.ds(base, _CHUNK)], xs, sem),
            pltpu.async_copy(p_hbm.at[pl.ds(base, _CHUNK)], ps, sem),
            pltpu.async_copy(t_hbm.at[pl.ds(base, _CHUNK)], ts, sem),
        )

    inflight = _issue(0, 0)
    for step in range(nsteps):
        slot = step % 2
        cur = inflight
        if step + 1 < nsteps:
            inflight = _issue(step + 1, 1 - slot)
        for c in cur:
            c.wait()
        j = step // _CHUNKS_PER_UNIT
        u = wid * _UNITS_PER_W + j
        ch = lax.rem(lax.div(u, 4), _NCH)
        laneoff = lane_iota + ch * (_NUM_BINS * _L)
        xs, ps, ts = bufs[slot]

        def _vec(v, _):
            sl = pl.ds(v * _L, _L)
            x = xs[sl]
            d = ps[sl] - ts[sl]
            bi = (x * jnp.float32(_NUM_BINS)).astype(jnp.int32)
            bi = jnp.minimum(jnp.maximum(bi, 0), _NUM_BINS - 1)
            idx = laneoff + bi * _L
            plsc.addupdate_scatter(acc_s, [idx], d)
            plsc.addupdate_scatter(acc_c, [idx], ones)
            return 0
        lax.fori_loop(0, _CHUNK // _L, _vec, 0, unroll=8)

    giota = lane_iota * _L
    for k in range(_NSEG // _L):
        ts = zeros
        tc = zeros
        for lane in range(_L):
            gidx = giota + (k * _L * _L + lane)
            ts = ts + plsc.load_gather(acc_s, [gidx])
            tc = tc + plsc.load_gather(acc_c, [gidx])
        row_s[pl.ds(k * _L, _L)] = ts
        row_c[pl.ds(k * _L, _L)] = tc

    pltpu.sync_copy(row_s, sums_out.at[wid])
    pltpu.sync_copy(row_c, cnts_out.at[wid])


_sc_call = functools.partial(
    pl.kernel,
    out_type=(jax.ShapeDtypeStruct((_NW, _NSEG), jnp.float32),
              jax.ShapeDtypeStruct((_NW, _NSEG), jnp.float32)),
    mesh=plsc.VectorSubcoreMesh(core_axis_name="c", subcore_axis_name="s"),
    compiler_params=pltpu.CompilerParams(needs_layout_passes=False),
    scratch_types=(
        pltpu.VMEM((_CHUNK,), jnp.float32),
        pltpu.VMEM((_CHUNK,), jnp.float32),
        pltpu.VMEM((_CHUNK,), jnp.float32),
        pltpu.VMEM((_CHUNK,), jnp.float32),
        pltpu.VMEM((_CHUNK,), jnp.float32),
        pltpu.VMEM((_CHUNK,), jnp.float32),
        pltpu.VMEM((_ACC_SIZE,), jnp.float32),
        pltpu.VMEM((_ACC_SIZE,), jnp.float32),
        pltpu.VMEM((_NSEG,), jnp.float32),
        pltpu.VMEM((_NSEG,), jnp.float32),
        pltpu.SemaphoreType.DMA,
        pltpu.SemaphoreType.DMA,
    ),
)(_sc_body)


def _tc_body(s_ref, c_ref, o_ref):
    s = jnp.sum(s_ref[...], axis=0, keepdims=True)
    c = jnp.sum(c_ref[...], axis=0, keepdims=True)
    nonempty = c > 0.0
    val = jnp.where(nonempty, jnp.abs(s) / jnp.where(nonempty, c, 1.0), 0.0)
    o_ref[...] = jnp.full((1, 1), jnp.sum(val) / jnp.float32(_NSEG))


def kernel(pred, target, input_img):
    x = input_img.reshape(-1)
    p = pred.reshape(-1)
    t = target.reshape(-1)
    sums, cnts = _sc_call(x, p, t)
    loss = pl.pallas_call(
        _tc_body,
        out_shape=jax.ShapeDtypeStruct((1, 1), jnp.float32),
    )(sums, cnts)
    return loss[0, 0]
